# baseline (speedup 1.0000x reference)
import functools

import jax
import jax.numpy as jnp
from jax.experimental import pallas as pl
from jax.experimental.pallas import tpu as pltpu


def _router_kernel(x_ref, w1_ref, b1_ref, w2_ref, b2_ref,
                   logits_ref, gate_ref):
    h = jnp.dot(x_ref[...].astype(jnp.bfloat16), w1_ref[...],
                preferred_element_type=jnp.float32)
    h = h + b1_ref[...]
    h = h * jax.nn.sigmoid(h)
    logits = jnp.dot(h.astype(jnp.bfloat16), w2_ref[...],
                     preferred_element_type=jnp.float32) + b2_ref[...]
    logits_ref[...] = logits
    m = jnp.max(logits, axis=-1, keepdims=True)
    e = jnp.exp(logits - m)
    gate_ref[...] = e / jnp.sum(e, axis=-1, keepdims=True)


@functools.partial(jax.jit, static_argnames=("tm",))
def _router(flow_input, W1, b1, W2, b2, tm=512):
    tokens, d_model = flow_input.shape
    hidden, num_experts = W2.shape
    tm = min(tm, tokens)
    ni = tokens // tm

    W1 = W1.astype(jnp.bfloat16)
    W2 = W2.astype(jnp.bfloat16)
    b1_2d = b1.reshape(1, hidden)
    b2_2d = b2.reshape(1, num_experts)

    out_shapes = (
        jax.ShapeDtypeStruct((tokens, num_experts), jnp.float32),
        jax.ShapeDtypeStruct((tokens, num_experts), jnp.float32),
    )

    return pl.pallas_call(
        _router_kernel,
        grid=(ni,),
        in_specs=[
            pl.BlockSpec((tm, d_model), lambda i: (i, 0)),
            pl.BlockSpec((d_model, hidden), lambda i: (0, 0)),
            pl.BlockSpec((1, hidden), lambda i: (0, 0)),
            pl.BlockSpec((hidden, num_experts), lambda i: (0, 0)),
            pl.BlockSpec((1, num_experts), lambda i: (0, 0)),
        ],
        out_specs=[
            pl.BlockSpec((tm, num_experts), lambda i: (i, 0)),
            pl.BlockSpec((tm, num_experts), lambda i: (i, 0)),
        ],
        out_shape=out_shapes,
        compiler_params=pltpu.CompilerParams(
            dimension_semantics=("parallel",),
            vmem_limit_bytes=62 * 1024 * 1024,
        ),
    )(flow_input, W1, b1_2d, W2, b2_2d)


def kernel(flow_input, W1, b1, W2, b2):
    return _router(flow_input, W1, b1, W2, b2)


# speedup vs baseline: 1.0110x; 1.0110x over previous
import functools

import jax
import jax.numpy as jnp
from jax.experimental import pallas as pl
from jax.experimental.pallas import tpu as pltpu


def _router_kernel(x_ref, w1_ref, b1_ref, w2_ref, b2_ref,
                   logits_ref, gate_ref):
    h = jnp.dot(x_ref[...].astype(jnp.bfloat16), w1_ref[...],
                preferred_element_type=jnp.float32)
    h = h + b1_ref[...]
    h = h * jax.nn.sigmoid(h)
    logits = jnp.dot(h.astype(jnp.bfloat16), w2_ref[...],
                     preferred_element_type=jnp.float32) + b2_ref[...]
    logits_ref[...] = logits
    m = jnp.max(logits, axis=-1, keepdims=True)
    e = jnp.exp(logits - m)
    gate_ref[...] = e / jnp.sum(e, axis=-1, keepdims=True)


@functools.partial(jax.jit, static_argnames=("tm",))
def _router(flow_input, W1, b1, W2, b2, tm=256):
    tokens, d_model = flow_input.shape
    hidden, num_experts = W2.shape
    tm = min(tm, tokens)
    ni = tokens // tm

    W1 = W1.astype(jnp.bfloat16)
    W2 = W2.astype(jnp.bfloat16)
    b1_2d = b1.reshape(1, hidden)
    b2_2d = b2.reshape(1, num_experts)

    out_shapes = (
        jax.ShapeDtypeStruct((tokens, num_experts), jnp.float32),
        jax.ShapeDtypeStruct((tokens, num_experts), jnp.float32),
    )

    return pl.pallas_call(
        _router_kernel,
        grid=(ni,),
        in_specs=[
            pl.BlockSpec((tm, d_model), lambda i: (i, 0)),
            pl.BlockSpec((d_model, hidden), lambda i: (0, 0)),
            pl.BlockSpec((1, hidden), lambda i: (0, 0)),
            pl.BlockSpec((hidden, num_experts), lambda i: (0, 0)),
            pl.BlockSpec((1, num_experts), lambda i: (0, 0)),
        ],
        out_specs=[
            pl.BlockSpec((tm, num_experts), lambda i: (i, 0)),
            pl.BlockSpec((tm, num_experts), lambda i: (i, 0)),
        ],
        out_shape=out_shapes,
        compiler_params=pltpu.CompilerParams(
            dimension_semantics=("parallel",),
            vmem_limit_bytes=62 * 1024 * 1024,
        ),
    )(flow_input, W1, b1_2d, W2, b2_2d)


def kernel(flow_input, W1, b1, W2, b2):
    return _router(flow_input, W1, b1, W2, b2)


# speedup vs baseline: 1.0226x; 1.0115x over previous
import functools

import jax
import jax.numpy as jnp
from jax.experimental import pallas as pl
from jax.experimental.pallas import tpu as pltpu


def _router_kernel(x_ref, w1_ref, b1_ref, w2_ref, b2_ref,
                   logits_ref, gate_ref):
    hidden = w1_ref.shape[1]
    ch = hidden // 2
    xb = x_ref[...].astype(jnp.bfloat16)
    part = None
    for c in range(2):
        cols = pl.ds(c * ch, ch)
        h = jnp.dot(xb, w1_ref[:, cols], preferred_element_type=jnp.float32)
        h = h + b1_ref[:, cols]
        h = h * jax.nn.sigmoid(h)
        p = jnp.dot(h.astype(jnp.bfloat16), w2_ref[cols, :],
                    preferred_element_type=jnp.float32)
        part = p if part is None else part + p
    logits = part + b2_ref[...]
    logits_ref[...] = logits
    m = jnp.max(logits, axis=-1, keepdims=True)
    e = jnp.exp(logits - m)
    gate_ref[...] = e / jnp.sum(e, axis=-1, keepdims=True)


@functools.partial(jax.jit, static_argnames=("tm",))
def _router(flow_input, W1, b1, W2, b2, tm=512):
    tokens, d_model = flow_input.shape
    hidden, num_experts = W2.shape
    tm = min(tm, tokens)
    ni = tokens // tm

    W1 = W1.astype(jnp.bfloat16)
    W2 = W2.astype(jnp.bfloat16)
    b1_2d = b1.reshape(1, hidden)
    b2_2d = b2.reshape(1, num_experts)

    out_shapes = (
        jax.ShapeDtypeStruct((tokens, num_experts), jnp.float32),
        jax.ShapeDtypeStruct((tokens, num_experts), jnp.float32),
    )

    return pl.pallas_call(
        _router_kernel,
        grid=(ni,),
        in_specs=[
            pl.BlockSpec((tm, d_model), lambda i: (i, 0)),
            pl.BlockSpec((d_model, hidden), lambda i: (0, 0)),
            pl.BlockSpec((1, hidden), lambda i: (0, 0)),
            pl.BlockSpec((hidden, num_experts), lambda i: (0, 0)),
            pl.BlockSpec((1, num_experts), lambda i: (0, 0)),
        ],
        out_specs=[
            pl.BlockSpec((tm, num_experts), lambda i: (i, 0)),
            pl.BlockSpec((tm, num_experts), lambda i: (i, 0)),
        ],
        out_shape=out_shapes,
        compiler_params=pltpu.CompilerParams(
            dimension_semantics=("parallel",),
            vmem_limit_bytes=62 * 1024 * 1024,
        ),
    )(flow_input, W1, b1_2d, W2, b2_2d)


def kernel(flow_input, W1, b1, W2, b2):
    return _router(flow_input, W1, b1, W2, b2)


# speedup vs baseline: 1.0291x; 1.0064x over previous
import functools

import jax
import jax.numpy as jnp
from jax.experimental import pallas as pl
from jax.experimental.pallas import tpu as pltpu


def _router_kernel(x_ref, w1_ref, b1_ref, w2_ref, b2_ref,
                   logits_ref, gate_ref):
    h = jnp.dot(x_ref[...].astype(jnp.bfloat16), w1_ref[...],
                preferred_element_type=jnp.float32)
    h = h + b1_ref[...]
    h = h * jax.nn.sigmoid(h)
    logits = jnp.dot(h.astype(jnp.bfloat16), w2_ref[...],
                     preferred_element_type=jnp.float32) + b2_ref[...]
    logits_ref[...] = logits
    m = jnp.max(logits, axis=-1, keepdims=True)
    e = jnp.exp(logits - m)
    gate_ref[...] = e / jnp.sum(e, axis=-1, keepdims=True)


@functools.partial(jax.jit, static_argnames=("tm",))
def _router(flow_input, W1, b1, W2, b2, tm=512):
    tokens, d_model = flow_input.shape
    hidden, num_experts = W2.shape
    tm = min(tm, tokens)
    ni = tokens // tm

    W1 = W1.astype(jnp.bfloat16)
    W2 = W2.astype(jnp.bfloat16)
    b1_2d = b1.reshape(1, hidden)
    b2_2d = b2.reshape(1, num_experts)

    out_shapes = (
        jax.ShapeDtypeStruct((tokens, num_experts), jnp.float32),
        jax.ShapeDtypeStruct((tokens, num_experts), jnp.float32),
    )

    return pl.pallas_call(
        _router_kernel,
        grid=(ni,),
        in_specs=[
            pl.BlockSpec((tm, d_model), lambda i: (i, 0)),
            pl.BlockSpec((d_model, hidden), lambda i: (0, 0)),
            pl.BlockSpec((1, hidden), lambda i: (0, 0)),
            pl.BlockSpec((hidden, num_experts), lambda i: (0, 0)),
            pl.BlockSpec((1, num_experts), lambda i: (0, 0)),
        ],
        out_specs=[
            pl.BlockSpec((tm, num_experts), lambda i: (i, 0)),
            pl.BlockSpec((tm, num_experts), lambda i: (i, 0)),
        ],
        out_shape=out_shapes,
        compiler_params=pltpu.CompilerParams(
            dimension_semantics=("parallel",),
            vmem_limit_bytes=62 * 1024 * 1024,
        ),
    )(flow_input, W1, b1_2d, W2, b2_2d)


def kernel(flow_input, W1, b1, W2, b2):
    return _router(flow_input, W1, b1, W2, b2)
